# Initial kernel scaffold; baseline (speedup 1.0000x reference)
import jax
import jax.numpy as jnp
from jax.experimental import pallas as pl


def kernel(x, dt, src_idx, dst_idx, W_src, b_src, W_dst, b_dst, attn_l_w, attn_l_b, attn_r_w, attn_r_b, time_w, time_b):
    raise NotImplementedError("write your pallas kernel here")


# speedup vs baseline: 14.6315x; 14.6315x over previous
import functools

import jax
import jax.numpy as jnp
from jax import lax
from jax.experimental import pallas as pl
from jax.experimental.pallas import tpu as pltpu
from jax.experimental.pallas import tpu_sc as plsc

N_DST = 10000
N_E = 160000
N_SRC = N_DST + N_E
D = 128
DT = 100
NH = 8
HW = 16
EPS = 1e-15
MAXN = 1.0 - 1e-5
CLIP = 1.0 - 1e-7

NC = 2
NS = 16
NW = NC * NS
CH = 64
EPT = 5120
NCHUNK = EPT // CH
E_PAD = NW * EPT
ACC_ROWS = 10112
ZPT = ACC_ROWS // NS
SUM_ROWS = 1280
SPT = SUM_ROWS // NS

BR = 1000


def _rnorm(v):
    return jnp.maximum(jnp.sqrt(jnp.sum(v * v, axis=-1, keepdims=True)), EPS)


def _artanh(z):
    z = jnp.clip(z, 0.0, CLIP)
    return 0.5 * jnp.log((1.0 + z) / (1.0 - z))


def _project_rows(v):
    n = _rnorm(v)
    return jnp.where(n > MAXN, MAXN / n, 1.0) * v


def _node_body(x_ref, t_ref, tw_ref, tb_ref, tm_ref, a_ref, b_ref, hb_ref,
               p_ref, ab_ref, fs_ref, el_ref):
    x = x_ref[...]
    t = t_ref[...]
    tf = jnp.cos(t * tw_ref[...] + tb_ref[...]) * tm_ref[...]
    xn = _rnorm(x)
    lx = _artanh(xn) * x / xn
    n1 = jnp.maximum(jnp.sqrt(jnp.sum(tf * tf, -1, keepdims=True)
                              + jnp.sum(lx * lx, -1, keepdims=True)), EPS)
    s1 = jnp.where(n1 > MAXN, MAXN / n1, 1.0)
    xn2 = jnp.maximum(n1 * s1, EPS)
    mx = (jnp.dot(tf, a_ref[...], preferred_element_type=jnp.float32)
          + jnp.dot(lx, b_ref[...], preferred_element_type=jnp.float32)) * s1
    mxn = _rnorm(mx)
    mm = jnp.tanh(mxn / xn2 * _artanh(xn2)) * mx / mxn
    res = _project_rows(mm)
    hb = hb_ref[...]
    x2 = jnp.sum(res * res, -1, keepdims=True)
    y2 = jnp.sum(hb * hb, -1, keepdims=True)
    xy = jnp.sum(res * hb, -1, keepdims=True)
    num = (1.0 + 2.0 * xy + y2) * res + (1.0 - x2) * hb
    den = 1.0 + 2.0 * xy + x2 * y2
    h3 = _project_rows(num / jnp.maximum(den, EPS))
    n3 = _rnorm(h3)
    fs = _artanh(n3) * h3 / n3
    fs_ref[...] = fs
    el_ref[...] = jnp.dot(fs, p_ref[...], preferred_element_type=jnp.float32) \
        + ab_ref[...]


def _node_stage(xx, tt, tw, tb, tm, a, b, hb, p, ab):
    rows = xx.shape[0]
    grid = rows // BR
    wspec = lambda shp: pl.BlockSpec(shp, lambda i: (0,) * len(shp))
    return pl.pallas_call(
        _node_body,
        grid=(grid,),
        in_specs=[
            pl.BlockSpec((BR, D), lambda i: (i, 0)),
            pl.BlockSpec((BR, 1), lambda i: (i, 0)),
            wspec((1, D)), wspec((1, D)), wspec((1, D)),
            wspec((D, D)), wspec((D, D)), wspec((1, D)),
            wspec((D, D)), wspec((1, 1)),
        ],
        out_specs=[
            pl.BlockSpec((BR, D), lambda i: (i, 0)),
            pl.BlockSpec((BR, D), lambda i: (i, 0)),
        ],
        out_shape=[
            jax.ShapeDtypeStruct((rows, D), jnp.float32),
            jax.ShapeDtypeStruct((rows, D), jnp.float32),
        ],
    )(xx, tt, tw, tb, tm, a, b, hb, p, ab)


def _edge_body(fs_hbm, el_hbm, er_hbm, srcp, dstp, out_hbm, sums_hbm,
               si_v, di_v, dp_v, elv, erv, wv, acc_sh, sums_sh):
    c = lax.axis_index("c")
    s = lax.axis_index("s")
    wid = s * NC + c
    z16 = jnp.zeros((HW,), jnp.float32)

    def _zero(i, _):
        for j in range(NH):
            elv[i, pl.ds(j * HW, HW)] = z16
            wv[i, pl.ds(j * HW, HW)] = z16
        return 0
    lax.fori_loop(0, CH, _zero, 0)
    for off in range(0, ZPT, CH):
        sz = min(CH, ZPT - off)
        pltpu.sync_copy(elv.at[pl.ds(0, sz)],
                        acc_sh.at[pl.ds(s * ZPT + off, sz)])
    for off in range(0, SPT, CH):
        sz = min(CH, SPT - off)
        pltpu.sync_copy(wv.at[pl.ds(0, sz)],
                        sums_sh.at[pl.ds(s * SPT + off, sz)])
    plsc.subcore_barrier()

    def _chunk(k, _):
        base = wid * EPT + k * CH
        pltpu.sync_copy(srcp.at[pl.ds(base, CH)], si_v)
        pltpu.sync_copy(dstp.at[pl.ds(base, CH)], di_v)
        pltpu.sync_copy(el_hbm.at[si_v], elv)
        pltpu.sync_copy(er_hbm.at[di_v], erv)

        def _dpack(i, _):
            dp_v[pl.ds(i * HW, HW)] = lax.shift_right_logical(
                di_v[pl.ds(i * HW, HW)], 3)
            return 0
        lax.fori_loop(0, CH // HW, _dpack, 0)

        def _wcalc(g, _):
            dvec = di_v[pl.ds(g * HW, HW)]
            for j in range(HW):
                i = g * HW + j
                e = elv[i, pl.ds(0, HW)] + erv[i, pl.ds(0, HW)]
                w = jnp.exp(jnp.where(e >= 0.0, e, 0.2 * e))
                for jj in range(NH):
                    wv[i, pl.ds(jj * HW, HW)] = z16
                lane = (dvec[j] & 7) * HW
                wv[i, pl.ds(lane, HW)] = w
            return 0
        lax.fori_loop(0, CH // HW, _wcalc, 0)
        pltpu.sync_copy(fs_hbm.at[si_v], elv)

        def _edge(g, _):
            dvec = di_v[pl.ds(g * HW, HW)]
            for j in range(HW):
                i = g * HW + j
                lane = (dvec[j] & 7) * HW
                w = wv[i, pl.ds(lane, HW)]
                for h in range(NH):
                    f = elv[i, pl.ds(h * HW, HW)]
                    elv[i, pl.ds(h * HW, HW)] = f * jnp.broadcast_to(w[h], (HW,))
            return 0
        lax.fori_loop(0, CH // HW, _edge, 0)
        pltpu.sync_copy(elv, acc_sh.at[di_v], add=True)
        pltpu.sync_copy(wv, sums_sh.at[dp_v], add=True)
        return 0
    lax.fori_loop(0, NCHUNK, _chunk, 0)
    plsc.subcore_barrier()

    pltpu.sync_copy(acc_sh.at[pl.ds(s * ZPT, ZPT)],
                    out_hbm.at[c, pl.ds(s * ZPT, ZPT)])
    pltpu.sync_copy(sums_sh.at[pl.ds(s * SPT, SPT)],
                    sums_hbm.at[c, pl.ds(s * SPT, SPT)])


@functools.cache
def _edge_stage():
    return pl.kernel(
        _edge_body,
        out_type=[
            jax.ShapeDtypeStruct((NC, ACC_ROWS, D), jnp.float32),
            jax.ShapeDtypeStruct((NC, SUM_ROWS, D), jnp.float32),
        ],
        mesh=plsc.VectorSubcoreMesh(core_axis_name="c", subcore_axis_name="s",
                                    num_cores=NC, num_subcores=NS),
        scratch_types=[
            pltpu.VMEM((CH,), jnp.int32),
            pltpu.VMEM((CH,), jnp.int32),
            pltpu.VMEM((CH,), jnp.int32),
            pltpu.VMEM((CH, D), jnp.float32),
            pltpu.VMEM((CH, D), jnp.float32),
            pltpu.VMEM((CH, D), jnp.float32),
            pltpu.VMEM_SHARED((ACC_ROWS, D), jnp.float32),
            pltpu.VMEM_SHARED((SUM_ROWS, D), jnp.float32),
        ],
    )


def _final_body(p_ref, s_ref, q_ref, o_ref):
    raw = p_ref[0] + p_ref[1]
    sums = s_ref[0] + s_ref[1]
    recip = 1.0 / jnp.maximum(sums, EPS)
    rst = raw * jnp.dot(recip, q_ref[...], preferred_element_type=jnp.float32)
    n = _rnorm(rst)
    e1 = _project_rows(jnp.tanh(n) * rst / n)
    n2 = _rnorm(e1)
    xt = jax.nn.relu(_artanh(n2) * e1 / n2)
    n3 = _rnorm(xt)
    o_ref[...] = _project_rows(jnp.tanh(n3) * xt / n3)


def _final_stage(p, sums, q):
    return pl.pallas_call(
        _final_body,
        grid=(N_DST // BR,),
        in_specs=[
            pl.BlockSpec((NC, BR, D), lambda i: (0, i, 0)),
            pl.BlockSpec((NC, BR, HW), lambda i: (0, i, 0)),
            pl.BlockSpec((HW, D), lambda i: (0, 0)),
        ],
        out_specs=pl.BlockSpec((BR, D), lambda i: (i, 0)),
        out_shape=jax.ShapeDtypeStruct((N_DST, D), jnp.float32),
    )(p, sums, q)


def _expmap0(u):
    n = jnp.maximum(jnp.sqrt(jnp.sum(u * u)), EPS)
    return jnp.tanh(n) * u / n


def _project_vec(v):
    n = jnp.maximum(jnp.sqrt(jnp.sum(v * v)), EPS)
    return jnp.where(n > MAXN, v / n * MAXN, v)


def kernel(x, dt, src_idx, dst_idx, W_src, b_src, W_dst, b_dst,
           attn_l_w, attn_l_b, attn_r_w, attn_r_b, time_w, time_b):
    f32 = jnp.float32
    t = jnp.concatenate([jnp.zeros((N_DST,), f32), dt]).reshape(-1, 1)
    tw = jnp.zeros((1, D), f32).at[0, :DT].set(time_w)
    tb = jnp.zeros((1, D), f32).at[0, :DT].set(time_b)
    tm = jnp.zeros((1, D), f32).at[0, :DT].set(1.0)
    a_s = jnp.zeros((D, D), f32).at[:DT].set(W_src[:, :DT].T)
    b_s = W_src[:, DT:].T
    a_d = jnp.zeros((D, D), f32).at[:DT].set(W_dst[:, :DT].T)
    b_d = W_dst[:, DT:].T
    hb_s = _project_vec(_expmap0(b_src)).reshape(1, D)
    hb_d = _project_vec(_expmap0(b_dst)).reshape(1, D)
    p_l = jnp.concatenate(
        [jnp.kron(jnp.eye(NH, dtype=f32), attn_l_w.reshape(HW, 1)),
         jnp.zeros((D, D - NH), f32)], axis=1)
    p_r = jnp.concatenate(
        [jnp.kron(jnp.eye(NH, dtype=f32), attn_r_w.reshape(HW, 1)),
         jnp.zeros((D, D - NH), f32)], axis=1)
    ab_l = attn_l_b.reshape(1, 1)
    ab_r = attn_r_b.reshape(1, 1)
    q = jnp.concatenate(
        [jnp.kron(jnp.eye(NH, dtype=f32), jnp.ones((1, HW), f32)),
         jnp.zeros((NH, D), f32)], axis=0)
    pad = E_PAD - N_E
    srcp = jnp.concatenate([src_idx.astype(jnp.int32),
                            jnp.zeros((pad,), jnp.int32)])
    dstp = jnp.concatenate([dst_idx.astype(jnp.int32),
                            jnp.full((pad,), N_DST + 8, jnp.int32)])

    fs, el = _node_stage(x, t, tw, tb, tm, a_s, b_s, hb_s, p_l, ab_l)
    _, er = _node_stage(x[:N_DST], t[:N_DST], tw, tb, tm, a_d, b_d, hb_d,
                        p_r, ab_r)

    part, sums = _edge_stage()(fs, el, er, srcp, dstp)

    sums16 = sums.reshape(NC, SUM_ROWS * 8, HW)[:, :N_DST]
    return _final_stage(part[:, :N_DST], sums16, q)


# speedup vs baseline: 16.9251x; 1.1568x over previous
import functools

import jax
import jax.numpy as jnp
from jax import lax
from jax.experimental import pallas as pl
from jax.experimental.pallas import tpu as pltpu
from jax.experimental.pallas import tpu_sc as plsc

N_DST = 10000
N_E = 160000
N_SRC = N_DST + N_E
D = 128
DT = 100
NH = 8
HW = 16
EPS = 1e-15
MAXN = 1.0 - 1e-5
CLIP = 1.0 - 1e-7

NC = 2
NS = 16
NW = NC * NS
CH = 80
EPT = 5120
NCHUNK = EPT // CH
E_PAD = NW * EPT
ACC_ROWS = 10112
ZPT = ACC_ROWS // NS
SUM_ROWS = 1280
SPT = SUM_ROWS // NS

BR = 1000


def _rnorm(v):
    return jnp.maximum(jnp.sqrt(jnp.sum(v * v, axis=-1, keepdims=True)), EPS)


def _artanh(z):
    z = jnp.clip(z, 0.0, CLIP)
    return 0.5 * jnp.log((1.0 + z) / (1.0 - z))


def _project_rows(v):
    n = _rnorm(v)
    return jnp.where(n > MAXN, MAXN / n, 1.0) * v


def _node_body(x_ref, t_ref, tw_ref, tb_ref, tm_ref, a_ref, b_ref, hb_ref,
               p_ref, ab_ref, fs_ref, el_ref):
    x = x_ref[...]
    t = t_ref[...]
    tf = jnp.cos(t * tw_ref[...] + tb_ref[...]) * tm_ref[...]
    xn = _rnorm(x)
    lx = _artanh(xn) * x / xn
    n1 = jnp.maximum(jnp.sqrt(jnp.sum(tf * tf, -1, keepdims=True)
                              + jnp.sum(lx * lx, -1, keepdims=True)), EPS)
    s1 = jnp.where(n1 > MAXN, MAXN / n1, 1.0)
    xn2 = jnp.maximum(n1 * s1, EPS)
    mx = (jnp.dot(tf, a_ref[...], preferred_element_type=jnp.float32)
          + jnp.dot(lx, b_ref[...], preferred_element_type=jnp.float32)) * s1
    mxn = _rnorm(mx)
    mm = jnp.tanh(mxn / xn2 * _artanh(xn2)) * mx / mxn
    res = _project_rows(mm)
    hb = hb_ref[...]
    x2 = jnp.sum(res * res, -1, keepdims=True)
    y2 = jnp.sum(hb * hb, -1, keepdims=True)
    xy = jnp.sum(res * hb, -1, keepdims=True)
    num = (1.0 + 2.0 * xy + y2) * res + (1.0 - x2) * hb
    den = 1.0 + 2.0 * xy + x2 * y2
    h3 = _project_rows(num / jnp.maximum(den, EPS))
    n3 = _rnorm(h3)
    fs = _artanh(n3) * h3 / n3
    fs_ref[...] = fs
    el_ref[...] = jnp.dot(fs, p_ref[...], preferred_element_type=jnp.float32) \
        + ab_ref[...]


def _node_stage(xx, tt, tw, tb, tm, a, b, hb, p, ab):
    rows = xx.shape[0]
    grid = rows // BR
    wspec = lambda shp: pl.BlockSpec(shp, lambda i: (0,) * len(shp))
    return pl.pallas_call(
        _node_body,
        grid=(grid,),
        in_specs=[
            pl.BlockSpec((BR, D), lambda i: (i, 0)),
            pl.BlockSpec((BR, 1), lambda i: (i, 0)),
            wspec((1, D)), wspec((1, D)), wspec((1, D)),
            wspec((D, D)), wspec((D, D)), wspec((1, D)),
            wspec((D, D)), wspec((1, 1)),
        ],
        out_specs=[
            pl.BlockSpec((BR, D), lambda i: (i, 0)),
            pl.BlockSpec((BR, D), lambda i: (i, 0)),
        ],
        out_shape=[
            jax.ShapeDtypeStruct((rows, D), jnp.float32),
            jax.ShapeDtypeStruct((rows, D), jnp.float32),
        ],
    )(xx, tt, tw, tb, tm, a, b, hb, p, ab)


def _edge_body(fs_hbm, el_hbm, er_hbm, srcp, dstp, out_hbm, sums_hbm,
               si_v, di_v, dp_v, elv, wv, sem1, sem2, acc_sh, sums_sh):
    c = lax.axis_index("c")
    s = lax.axis_index("s")
    wid = s * NC + c
    z16 = jnp.zeros((HW,), jnp.float32)

    def _zero(i, _):
        for j in range(NH):
            elv[i, pl.ds(j * HW, HW)] = z16
            wv[i, pl.ds(j * HW, HW)] = z16
        return 0
    lax.fori_loop(0, CH, _zero, 0)
    for off in range(0, ZPT, CH):
        sz = min(CH, ZPT - off)
        pltpu.sync_copy(elv.at[pl.ds(0, sz)],
                        acc_sh.at[pl.ds(s * ZPT + off, sz)])
    for off in range(0, SPT, CH):
        sz = min(CH, SPT - off)
        pltpu.sync_copy(wv.at[pl.ds(0, sz)],
                        sums_sh.at[pl.ds(s * SPT + off, sz)])
    plsc.subcore_barrier()

    def _chunk(k, _):
        base = wid * EPT + k * CH
        a = pltpu.async_copy(srcp.at[pl.ds(base, CH)], si_v, sem1)
        b = pltpu.async_copy(dstp.at[pl.ds(base, CH)], di_v, sem2)
        a.wait()
        b.wait()
        a = pltpu.async_copy(el_hbm.at[si_v], elv, sem1)
        b = pltpu.async_copy(er_hbm.at[di_v], wv, sem2)
        a.wait()
        b.wait()

        def _wcalc(g, _):
            dvec = di_v[pl.ds(g * HW, HW)]
            for j in range(HW):
                i = g * HW + j
                e = elv[i, pl.ds(0, HW)] + wv[i, pl.ds(0, HW)]
                w = jnp.exp(jnp.where(e >= 0.0, e, 0.2 * e))
                for jj in range(NH):
                    wv[i, pl.ds(jj * HW, HW)] = z16
                lane = (dvec[j] & 7) * HW
                wv[i, pl.ds(lane, HW)] = w
            return 0
        lax.fori_loop(0, CH // HW, _wcalc, 0)
        f = pltpu.async_copy(fs_hbm.at[si_v], elv, sem1)

        def _dpack(i, _):
            dp_v[pl.ds(i * HW, HW)] = lax.shift_right_logical(
                di_v[pl.ds(i * HW, HW)], 3)
            return 0
        lax.fori_loop(0, CH // HW, _dpack, 0)
        f.wait()

        def _edge(g, _):
            dvec = di_v[pl.ds(g * HW, HW)]
            for j in range(HW):
                i = g * HW + j
                lane = (dvec[j] & 7) * HW
                w = wv[i, pl.ds(lane, HW)]
                for h in range(NH):
                    f = elv[i, pl.ds(h * HW, HW)]
                    elv[i, pl.ds(h * HW, HW)] = f * jnp.broadcast_to(w[h], (HW,))
            return 0
        lax.fori_loop(0, CH // HW, _edge, 0)
        a = pltpu.async_copy(elv, acc_sh.at[di_v], sem1, add=True)
        b = pltpu.async_copy(wv, sums_sh.at[dp_v], sem2, add=True)
        a.wait()
        b.wait()
        return 0
    lax.fori_loop(0, NCHUNK, _chunk, 0)
    plsc.subcore_barrier()

    pltpu.sync_copy(acc_sh.at[pl.ds(s * ZPT, ZPT)],
                    out_hbm.at[c, pl.ds(s * ZPT, ZPT)])
    pltpu.sync_copy(sums_sh.at[pl.ds(s * SPT, SPT)],
                    sums_hbm.at[c, pl.ds(s * SPT, SPT)])


@functools.cache
def _edge_stage():
    return pl.kernel(
        _edge_body,
        out_type=[
            jax.ShapeDtypeStruct((NC, ACC_ROWS, D), jnp.float32),
            jax.ShapeDtypeStruct((NC, SUM_ROWS, D), jnp.float32),
        ],
        mesh=plsc.VectorSubcoreMesh(core_axis_name="c", subcore_axis_name="s",
                                    num_cores=NC, num_subcores=NS),
        scratch_types=[
            pltpu.VMEM((CH,), jnp.int32),
            pltpu.VMEM((CH,), jnp.int32),
            pltpu.VMEM((CH,), jnp.int32),
            pltpu.VMEM((CH, D), jnp.float32),
            pltpu.VMEM((CH, D), jnp.float32),
            pltpu.SemaphoreType.DMA,
            pltpu.SemaphoreType.DMA,
            pltpu.VMEM_SHARED((ACC_ROWS, D), jnp.float32),
            pltpu.VMEM_SHARED((SUM_ROWS, D), jnp.float32),
        ],
    )


def _final_body(p_ref, s_ref, q_ref, o_ref):
    raw = p_ref[0] + p_ref[1]
    sums = s_ref[0] + s_ref[1]
    recip = 1.0 / jnp.maximum(sums, EPS)
    rst = raw * jnp.dot(recip, q_ref[...], preferred_element_type=jnp.float32)
    n = _rnorm(rst)
    e1 = _project_rows(jnp.tanh(n) * rst / n)
    n2 = _rnorm(e1)
    xt = jax.nn.relu(_artanh(n2) * e1 / n2)
    n3 = _rnorm(xt)
    o_ref[...] = _project_rows(jnp.tanh(n3) * xt / n3)


def _final_stage(p, sums, q):
    return pl.pallas_call(
        _final_body,
        grid=(N_DST // BR,),
        in_specs=[
            pl.BlockSpec((NC, BR, D), lambda i: (0, i, 0)),
            pl.BlockSpec((NC, BR, HW), lambda i: (0, i, 0)),
            pl.BlockSpec((HW, D), lambda i: (0, 0)),
        ],
        out_specs=pl.BlockSpec((BR, D), lambda i: (i, 0)),
        out_shape=jax.ShapeDtypeStruct((N_DST, D), jnp.float32),
    )(p, sums, q)


def _expmap0(u):
    n = jnp.maximum(jnp.sqrt(jnp.sum(u * u)), EPS)
    return jnp.tanh(n) * u / n


def _project_vec(v):
    n = jnp.maximum(jnp.sqrt(jnp.sum(v * v)), EPS)
    return jnp.where(n > MAXN, v / n * MAXN, v)


def kernel(x, dt, src_idx, dst_idx, W_src, b_src, W_dst, b_dst,
           attn_l_w, attn_l_b, attn_r_w, attn_r_b, time_w, time_b):
    f32 = jnp.float32
    t = jnp.concatenate([jnp.zeros((N_DST,), f32), dt]).reshape(-1, 1)
    tw = jnp.zeros((1, D), f32).at[0, :DT].set(time_w)
    tb = jnp.zeros((1, D), f32).at[0, :DT].set(time_b)
    tm = jnp.zeros((1, D), f32).at[0, :DT].set(1.0)
    a_s = jnp.zeros((D, D), f32).at[:DT].set(W_src[:, :DT].T)
    b_s = W_src[:, DT:].T
    a_d = jnp.zeros((D, D), f32).at[:DT].set(W_dst[:, :DT].T)
    b_d = W_dst[:, DT:].T
    hb_s = _project_vec(_expmap0(b_src)).reshape(1, D)
    hb_d = _project_vec(_expmap0(b_dst)).reshape(1, D)
    p_l = jnp.concatenate(
        [jnp.kron(jnp.eye(NH, dtype=f32), attn_l_w.reshape(HW, 1)),
         jnp.zeros((D, D - NH), f32)], axis=1)
    p_r = jnp.concatenate(
        [jnp.kron(jnp.eye(NH, dtype=f32), attn_r_w.reshape(HW, 1)),
         jnp.zeros((D, D - NH), f32)], axis=1)
    ab_l = attn_l_b.reshape(1, 1)
    ab_r = attn_r_b.reshape(1, 1)
    q = jnp.concatenate(
        [jnp.kron(jnp.eye(NH, dtype=f32), jnp.ones((1, HW), f32)),
         jnp.zeros((NH, D), f32)], axis=0)
    pad = E_PAD - N_E
    srcp = jnp.concatenate([src_idx.astype(jnp.int32),
                            jnp.zeros((pad,), jnp.int32)])
    dstp = jnp.concatenate([dst_idx.astype(jnp.int32),
                            jnp.full((pad,), N_DST + 8, jnp.int32)])

    fs, el = _node_stage(x, t, tw, tb, tm, a_s, b_s, hb_s, p_l, ab_l)
    _, er = _node_stage(x[:N_DST], t[:N_DST], tw, tb, tm, a_d, b_d, hb_d,
                        p_r, ab_r)

    part, sums = _edge_stage()(fs, el, er, srcp, dstp)

    sums16 = sums.reshape(NC, SUM_ROWS * 8, HW)[:, :N_DST]
    return _final_stage(part[:, :N_DST], sums16, q)


# speedup vs baseline: 17.2720x; 1.0205x over previous
import functools

import jax
import jax.numpy as jnp
from jax import lax
from jax.experimental import pallas as pl
from jax.experimental.pallas import tpu as pltpu
from jax.experimental.pallas import tpu_sc as plsc

N_DST = 10000
N_E = 160000
N_SRC = N_DST + N_E
D = 128
DT = 100
NH = 8
HW = 16
EPS = 1e-15
MAXN = 1.0 - 1e-5
CLIP = 1.0 - 1e-7

NC = 2
NS = 16
NW = NC * NS
CH = 128
EPT = 5120
NCHUNK = EPT // CH
E_PAD = NW * EPT
ACC_ROWS = 10112
ZPT = ACC_ROWS // NS
SUM_ROWS = 1280
SPT = SUM_ROWS // NS

BR = 1000


def _rnorm(v):
    return jnp.maximum(jnp.sqrt(jnp.sum(v * v, axis=-1, keepdims=True)), EPS)


def _artanh(z):
    z = jnp.clip(z, 0.0, CLIP)
    return 0.5 * jnp.log((1.0 + z) / (1.0 - z))


def _project_rows(v):
    n = _rnorm(v)
    return jnp.where(n > MAXN, MAXN / n, 1.0) * v


def _node_body(x_ref, t_ref, tw_ref, tb_ref, tm_ref, a_ref, b_ref, hb_ref,
               p_ref, ab_ref, fs_ref, el_ref):
    x = x_ref[...]
    t = t_ref[...]
    tf = jnp.cos(t * tw_ref[...] + tb_ref[...]) * tm_ref[...]
    xn = _rnorm(x)
    lx = _artanh(xn) * x / xn
    n1 = jnp.maximum(jnp.sqrt(jnp.sum(tf * tf, -1, keepdims=True)
                              + jnp.sum(lx * lx, -1, keepdims=True)), EPS)
    s1 = jnp.where(n1 > MAXN, MAXN / n1, 1.0)
    xn2 = jnp.maximum(n1 * s1, EPS)
    mx = (jnp.dot(tf, a_ref[...], preferred_element_type=jnp.float32)
          + jnp.dot(lx, b_ref[...], preferred_element_type=jnp.float32)) * s1
    mxn = _rnorm(mx)
    mm = jnp.tanh(mxn / xn2 * _artanh(xn2)) * mx / mxn
    res = _project_rows(mm)
    hb = hb_ref[...]
    x2 = jnp.sum(res * res, -1, keepdims=True)
    y2 = jnp.sum(hb * hb, -1, keepdims=True)
    xy = jnp.sum(res * hb, -1, keepdims=True)
    num = (1.0 + 2.0 * xy + y2) * res + (1.0 - x2) * hb
    den = 1.0 + 2.0 * xy + x2 * y2
    h3 = _project_rows(num / jnp.maximum(den, EPS))
    n3 = _rnorm(h3)
    fs = _artanh(n3) * h3 / n3
    fs_ref[...] = fs
    el_ref[...] = jnp.dot(fs, p_ref[...], preferred_element_type=jnp.float32) \
        + ab_ref[...]


def _node_stage(xx, tt, tw, tb, tm, a, b, hb, p, ab):
    rows = xx.shape[0]
    grid = rows // BR
    wspec = lambda shp: pl.BlockSpec(shp, lambda i: (0,) * len(shp))
    return pl.pallas_call(
        _node_body,
        grid=(grid,),
        in_specs=[
            pl.BlockSpec((BR, D), lambda i: (i, 0)),
            pl.BlockSpec((BR, 1), lambda i: (i, 0)),
            wspec((1, D)), wspec((1, D)), wspec((1, D)),
            wspec((D, D)), wspec((D, D)), wspec((1, D)),
            wspec((D, D)), wspec((1, 1)),
        ],
        out_specs=[
            pl.BlockSpec((BR, D), lambda i: (i, 0)),
            pl.BlockSpec((BR, D), lambda i: (i, 0)),
        ],
        out_shape=[
            jax.ShapeDtypeStruct((rows, D), jnp.float32),
            jax.ShapeDtypeStruct((rows, D), jnp.float32),
        ],
    )(xx, tt, tw, tb, tm, a, b, hb, p, ab)


def _edge_body(fs_hbm, el_hbm, er_hbm, srcp, dstp, out_hbm, sums_hbm,
               si_v, di_v, dp_v, elv, wv, sem1, sem2, acc_sh, sums_sh):
    c = lax.axis_index("c")
    s = lax.axis_index("s")
    wid = s * NC + c
    z16 = jnp.zeros((HW,), jnp.float32)

    def _zero(i, _):
        for j in range(NH):
            elv[i, pl.ds(j * HW, HW)] = z16
            wv[i, pl.ds(j * HW, HW)] = z16
        return 0
    lax.fori_loop(0, CH, _zero, 0)
    for off in range(0, ZPT, CH):
        sz = min(CH, ZPT - off)
        pltpu.sync_copy(elv.at[pl.ds(0, sz)],
                        acc_sh.at[pl.ds(s * ZPT + off, sz)])
    for off in range(0, SPT, CH):
        sz = min(CH, SPT - off)
        pltpu.sync_copy(wv.at[pl.ds(0, sz)],
                        sums_sh.at[pl.ds(s * SPT + off, sz)])
    plsc.subcore_barrier()

    def _chunk(k, _):
        base = wid * EPT + k * CH
        a = pltpu.async_copy(srcp.at[pl.ds(base, CH)], si_v, sem1)
        b = pltpu.async_copy(dstp.at[pl.ds(base, CH)], di_v, sem2)
        a.wait()
        b.wait()
        a = pltpu.async_copy(el_hbm.at[si_v], elv, sem1)
        b = pltpu.async_copy(er_hbm.at[di_v], wv, sem2)
        a.wait()
        b.wait()

        def _wcalc(g, _):
            dvec = di_v[pl.ds(g * HW, HW)]
            for j in range(HW):
                i = g * HW + j
                e = elv[i, pl.ds(0, HW)] + wv[i, pl.ds(0, HW)]
                w = jnp.exp(jnp.where(e >= 0.0, e, 0.2 * e))
                for jj in range(NH):
                    wv[i, pl.ds(jj * HW, HW)] = z16
                lane = (dvec[j] & 7) * HW
                wv[i, pl.ds(lane, HW)] = w
            return 0
        lax.fori_loop(0, CH // HW, _wcalc, 0)
        f = pltpu.async_copy(fs_hbm.at[si_v], elv, sem1)

        def _dpack(i, _):
            dp_v[pl.ds(i * HW, HW)] = lax.shift_right_logical(
                di_v[pl.ds(i * HW, HW)], 3)
            return 0
        lax.fori_loop(0, CH // HW, _dpack, 0)
        f.wait()

        def _edge(g, _):
            dvec = di_v[pl.ds(g * HW, HW)]
            for j in range(HW):
                i = g * HW + j
                lane = (dvec[j] & 7) * HW
                w = wv[i, pl.ds(lane, HW)]
                for h in range(NH):
                    f = elv[i, pl.ds(h * HW, HW)]
                    elv[i, pl.ds(h * HW, HW)] = f * jnp.broadcast_to(w[h], (HW,))
            return 0
        lax.fori_loop(0, CH // HW, _edge, 0)
        a = pltpu.async_copy(elv, acc_sh.at[di_v], sem1, add=True)
        b = pltpu.async_copy(wv, sums_sh.at[dp_v], sem2, add=True)
        a.wait()
        b.wait()
        return 0
    lax.fori_loop(0, NCHUNK, _chunk, 0)
    plsc.subcore_barrier()

    pltpu.sync_copy(acc_sh.at[pl.ds(s * ZPT, ZPT)],
                    out_hbm.at[c, pl.ds(s * ZPT, ZPT)])
    pltpu.sync_copy(sums_sh.at[pl.ds(s * SPT, SPT)],
                    sums_hbm.at[c, pl.ds(s * SPT, SPT)])


@functools.cache
def _edge_stage():
    return pl.kernel(
        _edge_body,
        out_type=[
            jax.ShapeDtypeStruct((NC, ACC_ROWS, D), jnp.float32),
            jax.ShapeDtypeStruct((NC, SUM_ROWS, D), jnp.float32),
        ],
        mesh=plsc.VectorSubcoreMesh(core_axis_name="c", subcore_axis_name="s",
                                    num_cores=NC, num_subcores=NS),
        scratch_types=[
            pltpu.VMEM((CH,), jnp.int32),
            pltpu.VMEM((CH,), jnp.int32),
            pltpu.VMEM((CH,), jnp.int32),
            pltpu.VMEM((CH, D), jnp.float32),
            pltpu.VMEM((CH, D), jnp.float32),
            pltpu.SemaphoreType.DMA,
            pltpu.SemaphoreType.DMA,
            pltpu.VMEM_SHARED((ACC_ROWS, D), jnp.float32),
            pltpu.VMEM_SHARED((SUM_ROWS, D), jnp.float32),
        ],
    )


def _final_body(p_ref, s_ref, q_ref, o_ref):
    raw = p_ref[0] + p_ref[1]
    sums = s_ref[0] + s_ref[1]
    recip = 1.0 / jnp.maximum(sums, EPS)
    rst = raw * jnp.dot(recip, q_ref[...], preferred_element_type=jnp.float32)
    n = _rnorm(rst)
    e1 = _project_rows(jnp.tanh(n) * rst / n)
    n2 = _rnorm(e1)
    xt = jax.nn.relu(_artanh(n2) * e1 / n2)
    n3 = _rnorm(xt)
    o_ref[...] = _project_rows(jnp.tanh(n3) * xt / n3)


def _final_stage(p, sums, q):
    return pl.pallas_call(
        _final_body,
        grid=(N_DST // BR,),
        in_specs=[
            pl.BlockSpec((NC, BR, D), lambda i: (0, i, 0)),
            pl.BlockSpec((NC, BR, HW), lambda i: (0, i, 0)),
            pl.BlockSpec((HW, D), lambda i: (0, 0)),
        ],
        out_specs=pl.BlockSpec((BR, D), lambda i: (i, 0)),
        out_shape=jax.ShapeDtypeStruct((N_DST, D), jnp.float32),
    )(p, sums, q)


def _expmap0(u):
    n = jnp.maximum(jnp.sqrt(jnp.sum(u * u)), EPS)
    return jnp.tanh(n) * u / n


def _project_vec(v):
    n = jnp.maximum(jnp.sqrt(jnp.sum(v * v)), EPS)
    return jnp.where(n > MAXN, v / n * MAXN, v)


def kernel(x, dt, src_idx, dst_idx, W_src, b_src, W_dst, b_dst,
           attn_l_w, attn_l_b, attn_r_w, attn_r_b, time_w, time_b):
    f32 = jnp.float32
    t = jnp.concatenate([jnp.zeros((N_DST,), f32), dt]).reshape(-1, 1)
    tw = jnp.zeros((1, D), f32).at[0, :DT].set(time_w)
    tb = jnp.zeros((1, D), f32).at[0, :DT].set(time_b)
    tm = jnp.zeros((1, D), f32).at[0, :DT].set(1.0)
    a_s = jnp.zeros((D, D), f32).at[:DT].set(W_src[:, :DT].T)
    b_s = W_src[:, DT:].T
    a_d = jnp.zeros((D, D), f32).at[:DT].set(W_dst[:, :DT].T)
    b_d = W_dst[:, DT:].T
    hb_s = _project_vec(_expmap0(b_src)).reshape(1, D)
    hb_d = _project_vec(_expmap0(b_dst)).reshape(1, D)
    p_l = jnp.concatenate(
        [jnp.kron(jnp.eye(NH, dtype=f32), attn_l_w.reshape(HW, 1)),
         jnp.zeros((D, D - NH), f32)], axis=1)
    p_r = jnp.concatenate(
        [jnp.kron(jnp.eye(NH, dtype=f32), attn_r_w.reshape(HW, 1)),
         jnp.zeros((D, D - NH), f32)], axis=1)
    ab_l = attn_l_b.reshape(1, 1)
    ab_r = attn_r_b.reshape(1, 1)
    q = jnp.concatenate(
        [jnp.kron(jnp.eye(NH, dtype=f32), jnp.ones((1, HW), f32)),
         jnp.zeros((NH, D), f32)], axis=0)
    pad = E_PAD - N_E
    srcp = jnp.concatenate([src_idx.astype(jnp.int32),
                            jnp.zeros((pad,), jnp.int32)])
    dstp = jnp.concatenate([dst_idx.astype(jnp.int32),
                            jnp.full((pad,), N_DST + 8, jnp.int32)])

    fs, el = _node_stage(x, t, tw, tb, tm, a_s, b_s, hb_s, p_l, ab_l)
    _, er = _node_stage(x[:N_DST], t[:N_DST], tw, tb, tm, a_d, b_d, hb_d,
                        p_r, ab_r)

    part, sums = _edge_stage()(fs, el, er, srcp, dstp)

    sums16 = sums.reshape(NC, SUM_ROWS * 8, HW)[:, :N_DST]
    return _final_stage(part[:, :N_DST], sums16, q)


# speedup vs baseline: 17.4535x; 1.0105x over previous
import functools

import jax
import jax.numpy as jnp
from jax import lax
from jax.experimental import pallas as pl
from jax.experimental.pallas import tpu as pltpu
from jax.experimental.pallas import tpu_sc as plsc

N_DST = 10000
N_E = 160000
N_SRC = N_DST + N_E
D = 128
DT = 100
NH = 8
HW = 16
EPS = 1e-15
MAXN = 1.0 - 1e-5
CLIP = 1.0 - 1e-7

NC = 2
NS = 16
NW = NC * NS
CH = 128
EPT = 5120
NCHUNK = EPT // CH
E_PAD = NW * EPT
ACC_ROWS = 10112
ZPT = ACC_ROWS // NS
SUM_ROWS = 1280
SPT = SUM_ROWS // NS

BR = 1000


def _rnorm(v):
    return jnp.maximum(jnp.sqrt(jnp.sum(v * v, axis=-1, keepdims=True)), EPS)


def _artanh(z):
    z = jnp.clip(z, 0.0, CLIP)
    return 0.5 * jnp.log((1.0 + z) / (1.0 - z))


def _project_rows(v):
    n = _rnorm(v)
    return jnp.where(n > MAXN, MAXN / n, 1.0) * v


def _node_body(x_ref, t_ref, tw_ref, tb_ref, tm_ref, a_ref, b_ref, hb_ref,
               p_ref, ab_ref, fs_ref, el_ref):
    x = x_ref[...]
    t = t_ref[...]
    tf = jnp.cos(t * tw_ref[...] + tb_ref[...]) * tm_ref[...]
    xn = _rnorm(x)
    lx = _artanh(xn) * x / xn
    n1 = jnp.maximum(jnp.sqrt(jnp.sum(tf * tf, -1, keepdims=True)
                              + jnp.sum(lx * lx, -1, keepdims=True)), EPS)
    s1 = jnp.where(n1 > MAXN, MAXN / n1, 1.0)
    xn2 = jnp.maximum(n1 * s1, EPS)
    mx = (jnp.dot(tf, a_ref[...], preferred_element_type=jnp.float32)
          + jnp.dot(lx, b_ref[...], preferred_element_type=jnp.float32)) * s1
    mxn = _rnorm(mx)
    mm = jnp.tanh(mxn / xn2 * _artanh(xn2)) * mx / mxn
    res = _project_rows(mm)
    hb = hb_ref[...]
    x2 = jnp.sum(res * res, -1, keepdims=True)
    y2 = jnp.sum(hb * hb, -1, keepdims=True)
    xy = jnp.sum(res * hb, -1, keepdims=True)
    num = (1.0 + 2.0 * xy + y2) * res + (1.0 - x2) * hb
    den = 1.0 + 2.0 * xy + x2 * y2
    h3 = _project_rows(num / jnp.maximum(den, EPS))
    n3 = _rnorm(h3)
    fs = _artanh(n3) * h3 / n3
    fs_ref[...] = fs
    el_ref[...] = jnp.dot(fs, p_ref[...], preferred_element_type=jnp.float32) \
        + ab_ref[...]


def _node_stage(xx, tt, tw, tb, tm, a, b, hb, p, ab):
    rows = xx.shape[0]
    grid = rows // BR
    wspec = lambda shp: pl.BlockSpec(shp, lambda i: (0,) * len(shp))
    return pl.pallas_call(
        _node_body,
        grid=(grid,),
        in_specs=[
            pl.BlockSpec((BR, D), lambda i: (i, 0)),
            pl.BlockSpec((BR, 1), lambda i: (i, 0)),
            wspec((1, D)), wspec((1, D)), wspec((1, D)),
            wspec((D, D)), wspec((D, D)), wspec((1, D)),
            wspec((D, D)), wspec((1, 1)),
        ],
        out_specs=[
            pl.BlockSpec((BR, D), lambda i: (i, 0)),
            pl.BlockSpec((BR, D), lambda i: (i, 0)),
        ],
        out_shape=[
            jax.ShapeDtypeStruct((rows, D), jnp.float32),
            jax.ShapeDtypeStruct((rows, D), jnp.float32),
        ],
    )(xx, tt, tw, tb, tm, a, b, hb, p, ab)


def _edge_body(fs_hbm, el_hbm, er_hbm, srcp, dstp, out_hbm, sums_hbm,
               si_v, di_v, dp_v, elv, wv, sem1, sem2, sem3, sem4,
               acc_sh, sums_sh):
    c = lax.axis_index("c")
    s = lax.axis_index("s")
    wid = s * NC + c
    z16 = jnp.zeros((HW,), jnp.float32)

    def _zero(i, _):
        for j in range(NH):
            elv[i, pl.ds(j * HW, HW)] = z16
            wv[i, pl.ds(j * HW, HW)] = z16
        return 0
    lax.fori_loop(0, CH, _zero, 0)
    for off in range(0, ZPT, CH):
        sz = min(CH, ZPT - off)
        pltpu.sync_copy(elv.at[pl.ds(0, sz)],
                        acc_sh.at[pl.ds(s * ZPT + off, sz)])
    for off in range(0, SPT, CH):
        sz = min(CH, SPT - off)
        pltpu.sync_copy(wv.at[pl.ds(0, sz)],
                        sums_sh.at[pl.ds(s * SPT + off, sz)])
    plsc.subcore_barrier()

    def _chunk(k, _):
        p = k & 1
        base = wid * EPT + k * CH
        a = pltpu.async_copy(srcp.at[pl.ds(base, CH)], si_v.at[p], sem3)
        b = pltpu.async_copy(dstp.at[pl.ds(base, CH)], di_v.at[p], sem4)

        @pl.when(k > 0)
        def _drain():
            q = 1 - p
            pltpu.make_async_copy(elv, acc_sh.at[di_v.at[q]], sem1).wait()
            pltpu.make_async_copy(wv, sums_sh.at[dp_v.at[q]], sem2).wait()
        a.wait()
        b.wait()
        a = pltpu.async_copy(el_hbm.at[si_v.at[p]], elv, sem1)
        b = pltpu.async_copy(er_hbm.at[di_v.at[p]], wv, sem2)
        a.wait()
        b.wait()

        def _wcalc(g, _):
            dvec = di_v[p, pl.ds(g * HW, HW)]
            for j in range(HW):
                i = g * HW + j
                e = elv[i, pl.ds(0, HW)] + wv[i, pl.ds(0, HW)]
                w = jnp.exp(jnp.where(e >= 0.0, e, 0.2 * e))
                for jj in range(NH):
                    wv[i, pl.ds(jj * HW, HW)] = z16
                lane = (dvec[j] & 7) * HW
                wv[i, pl.ds(lane, HW)] = w
            return 0
        lax.fori_loop(0, CH // HW, _wcalc, 0)
        f = pltpu.async_copy(fs_hbm.at[si_v.at[p]], elv, sem1)

        def _dpack(i, _):
            dp_v[p, pl.ds(i * HW, HW)] = lax.shift_right_logical(
                di_v[p, pl.ds(i * HW, HW)], 3)
            return 0
        lax.fori_loop(0, CH // HW, _dpack, 0)
        f.wait()

        def _edge(g, _):
            dvec = di_v[p, pl.ds(g * HW, HW)]
            for j in range(HW):
                i = g * HW + j
                lane = (dvec[j] & 7) * HW
                w = wv[i, pl.ds(lane, HW)]
                for h in range(NH):
                    f = elv[i, pl.ds(h * HW, HW)]
                    elv[i, pl.ds(h * HW, HW)] = f * jnp.broadcast_to(w[h], (HW,))
            return 0
        lax.fori_loop(0, CH // HW, _edge, 0)
        pltpu.async_copy(elv, acc_sh.at[di_v.at[p]], sem1, add=True)
        pltpu.async_copy(wv, sums_sh.at[dp_v.at[p]], sem2, add=True)
        return 0
    lax.fori_loop(0, NCHUNK, _chunk, 0)
    pltpu.make_async_copy(elv, acc_sh.at[di_v.at[(NCHUNK - 1) & 1]],
                          sem1).wait()
    pltpu.make_async_copy(wv, sums_sh.at[dp_v.at[(NCHUNK - 1) & 1]],
                          sem2).wait()
    plsc.subcore_barrier()

    pltpu.sync_copy(acc_sh.at[pl.ds(s * ZPT, ZPT)],
                    out_hbm.at[c, pl.ds(s * ZPT, ZPT)])
    pltpu.sync_copy(sums_sh.at[pl.ds(s * SPT, SPT)],
                    sums_hbm.at[c, pl.ds(s * SPT, SPT)])


@functools.cache
def _edge_stage():
    return pl.kernel(
        _edge_body,
        out_type=[
            jax.ShapeDtypeStruct((NC, ACC_ROWS, D), jnp.float32),
            jax.ShapeDtypeStruct((NC, SUM_ROWS, D), jnp.float32),
        ],
        mesh=plsc.VectorSubcoreMesh(core_axis_name="c", subcore_axis_name="s",
                                    num_cores=NC, num_subcores=NS),
        scratch_types=[
            pltpu.VMEM((2, CH), jnp.int32),
            pltpu.VMEM((2, CH), jnp.int32),
            pltpu.VMEM((2, CH), jnp.int32),
            pltpu.VMEM((CH, D), jnp.float32),
            pltpu.VMEM((CH, D), jnp.float32),
            pltpu.SemaphoreType.DMA,
            pltpu.SemaphoreType.DMA,
            pltpu.SemaphoreType.DMA,
            pltpu.SemaphoreType.DMA,
            pltpu.VMEM_SHARED((ACC_ROWS, D), jnp.float32),
            pltpu.VMEM_SHARED((SUM_ROWS, D), jnp.float32),
        ],
    )


def _final_body(p_ref, s_ref, q_ref, o_ref):
    raw = p_ref[0] + p_ref[1]
    sums = s_ref[0] + s_ref[1]
    recip = 1.0 / jnp.maximum(sums, EPS)
    rst = raw * jnp.dot(recip, q_ref[...], preferred_element_type=jnp.float32)
    n = _rnorm(rst)
    e1 = _project_rows(jnp.tanh(n) * rst / n)
    n2 = _rnorm(e1)
    xt = jax.nn.relu(_artanh(n2) * e1 / n2)
    n3 = _rnorm(xt)
    o_ref[...] = _project_rows(jnp.tanh(n3) * xt / n3)


def _final_stage(p, sums, q):
    return pl.pallas_call(
        _final_body,
        grid=(N_DST // BR,),
        in_specs=[
            pl.BlockSpec((NC, BR, D), lambda i: (0, i, 0)),
            pl.BlockSpec((NC, BR, HW), lambda i: (0, i, 0)),
            pl.BlockSpec((HW, D), lambda i: (0, 0)),
        ],
        out_specs=pl.BlockSpec((BR, D), lambda i: (i, 0)),
        out_shape=jax.ShapeDtypeStruct((N_DST, D), jnp.float32),
    )(p, sums, q)


def _expmap0(u):
    n = jnp.maximum(jnp.sqrt(jnp.sum(u * u)), EPS)
    return jnp.tanh(n) * u / n


def _project_vec(v):
    n = jnp.maximum(jnp.sqrt(jnp.sum(v * v)), EPS)
    return jnp.where(n > MAXN, v / n * MAXN, v)


def kernel(x, dt, src_idx, dst_idx, W_src, b_src, W_dst, b_dst,
           attn_l_w, attn_l_b, attn_r_w, attn_r_b, time_w, time_b):
    f32 = jnp.float32
    t = jnp.concatenate([jnp.zeros((N_DST,), f32), dt]).reshape(-1, 1)
    tw = jnp.zeros((1, D), f32).at[0, :DT].set(time_w)
    tb = jnp.zeros((1, D), f32).at[0, :DT].set(time_b)
    tm = jnp.zeros((1, D), f32).at[0, :DT].set(1.0)
    a_s = jnp.zeros((D, D), f32).at[:DT].set(W_src[:, :DT].T)
    b_s = W_src[:, DT:].T
    a_d = jnp.zeros((D, D), f32).at[:DT].set(W_dst[:, :DT].T)
    b_d = W_dst[:, DT:].T
    hb_s = _project_vec(_expmap0(b_src)).reshape(1, D)
    hb_d = _project_vec(_expmap0(b_dst)).reshape(1, D)
    p_l = jnp.concatenate(
        [jnp.kron(jnp.eye(NH, dtype=f32), attn_l_w.reshape(HW, 1)),
         jnp.zeros((D, D - NH), f32)], axis=1)
    p_r = jnp.concatenate(
        [jnp.kron(jnp.eye(NH, dtype=f32), attn_r_w.reshape(HW, 1)),
         jnp.zeros((D, D - NH), f32)], axis=1)
    ab_l = attn_l_b.reshape(1, 1)
    ab_r = attn_r_b.reshape(1, 1)
    q = jnp.concatenate(
        [jnp.kron(jnp.eye(NH, dtype=f32), jnp.ones((1, HW), f32)),
         jnp.zeros((NH, D), f32)], axis=0)
    pad = E_PAD - N_E
    srcp = jnp.concatenate([src_idx.astype(jnp.int32),
                            jnp.zeros((pad,), jnp.int32)])
    dstp = jnp.concatenate([dst_idx.astype(jnp.int32),
                            jnp.full((pad,), N_DST + 8, jnp.int32)])

    fs, el = _node_stage(x, t, tw, tb, tm, a_s, b_s, hb_s, p_l, ab_l)
    _, er = _node_stage(x[:N_DST], t[:N_DST], tw, tb, tm, a_d, b_d, hb_d,
                        p_r, ab_r)

    part, sums = _edge_stage()(fs, el, er, srcp, dstp)

    sums16 = sums.reshape(NC, SUM_ROWS * 8, HW)[:, :N_DST]
    return _final_stage(part[:, :N_DST], sums16, q)
